# baseline (speedup 1.0000x reference)
import jax
import jax.numpy as jnp
from jax import lax
from jax.experimental import pallas as pl
from jax.experimental.pallas import tpu as pltpu
from jax.experimental.pallas import tpu_sc as plsc

_C = 128
_NC, _NS = 2, 16
_NW = _NC * _NS
_CHUNK = 128
_NBUF = 4
_PF = 2


def _row_pass(buf, const_regs, col_idx, r):
    m = buf[r, pl.ds(0, 16)]
    for j in range(1, 8):
        m = jnp.maximum(m, buf[r, pl.ds(j * 16, 16)])
    n_le = plsc.all_reduce_population_count(m <= 1.0)
    n_eq = plsc.all_reduce_population_count(m == 1.0)
    overwrite16 = jnp.logical_or(n_le < 16, n_eq == 0)
    row_idx = jnp.full((16,), r, dtype=jnp.int32)
    for j in range(8):
        plsc.store_scatter(buf, [row_idx, col_idx[j]], const_regs[j],
                           mask=overwrite16)


def _sc_body(cube_hbm, const_hbm, out_hbm, bufs, constv, in_sems, out_sems):
    wid = lax.axis_index("s") * _NC + lax.axis_index("c")
    rows_per_w = cube_hbm.shape[0] // _NW
    base = wid * rows_per_w
    nchunk = rows_per_w // _CHUNK
    pltpu.sync_copy(const_hbm, constv)
    const_regs = [constv[pl.ds(j * 16, 16)] for j in range(8)]
    iota16 = lax.iota(jnp.int32, 16)
    col_idx = [iota16 + (j * 16) for j in range(8)]

    def in_slice(idx):
        return cube_hbm.at[pl.ds(base + idx * _CHUNK, _CHUNK), :]

    def out_slice(idx):
        return out_hbm.at[pl.ds(base + idx * _CHUNK, _CHUNK), :]

    for b in range(_PF):
        pltpu.async_copy(in_slice(b), bufs[b], in_sems[b])

    def super_body(i):
        for b in range(_NBUF):
            idx = i + b
            pf = idx + _PF
            bpf = (b + _PF) % _NBUF

            @pl.when(pf >= _NBUF)
            def _():
                pltpu.make_async_copy(
                    bufs[bpf], out_slice(0), out_sems[bpf]).wait()

            @pl.when(pf < nchunk)
            def _():
                pltpu.async_copy(in_slice(pf), bufs[bpf], in_sems[bpf])

            pltpu.make_async_copy(in_slice(0), bufs[b], in_sems[b]).wait()
            plsc.parallel_loop(0, _CHUNK, 1, unroll=8)(
                lambda r: _row_pass(bufs[b], const_regs, col_idx, r))
            pltpu.async_copy(bufs[b], out_slice(idx), out_sems[b])

    pl.loop(0, nchunk, step=_NBUF)(super_body)
    for b in range(_NBUF - _PF, _NBUF):
        pltpu.make_async_copy(bufs[b], out_slice(0), out_sems[b]).wait()


def kernel(cayley_cube, constant_distribution):
    b, n, _, c = cayley_cube.shape
    rows = b * n * n
    flat = cayley_cube.reshape(rows, c)
    mesh = plsc.VectorSubcoreMesh(
        core_axis_name="c", subcore_axis_name="s",
        num_cores=_NC, num_subcores=_NS,
    )
    out = pl.kernel(
        _sc_body,
        out_type=jax.ShapeDtypeStruct((rows, c), jnp.float32),
        mesh=mesh,
        compiler_params=pltpu.CompilerParams(needs_layout_passes=False),
        scratch_types=[
            [pltpu.VMEM((_CHUNK, c), jnp.float32) for _ in range(_NBUF)],
            pltpu.VMEM((c,), jnp.float32),
            [pltpu.SemaphoreType.DMA for _ in range(_NBUF)],
            [pltpu.SemaphoreType.DMA for _ in range(_NBUF)],
        ],
    )(flat, constant_distribution)
    return out.reshape(b, n, n, c)


# speedup vs baseline: 1.8568x; 1.8568x over previous
import jax
import jax.numpy as jnp
from jax import lax
from jax.experimental import pallas as pl
from jax.experimental.pallas import tpu as pltpu
from jax.experimental.pallas import tpu_sc as plsc

_C = 128
_NC, _NS = 2, 16
_NW = _NC * _NS
_CHUNK = 128
_NBUF = 4
_PF = 2
_SC_ROWS = 65536

_TC_BLOCK = 16384


def _row_pass(buf, const_regs, r):
    xs = [buf[r, pl.ds(j * 16, 16)] for j in range(8)]
    m = xs[0]
    for j in range(1, 8):
        m = jnp.maximum(m, xs[j])
    n_le = plsc.all_reduce_population_count(m <= 1.0)
    n_eq = plsc.all_reduce_population_count(m == 1.0)
    keep16 = jnp.logical_and(n_le == 16, n_eq > 0)
    for j in range(8):
        buf[r, pl.ds(j * 16, 16)] = jnp.where(keep16, xs[j], const_regs[j])


def _sc_body(cube_hbm, const_hbm, out_hbm, bufs, constv, in_sems, out_sems):
    wid = lax.axis_index("s") * _NC + lax.axis_index("c")
    rows_per_w = out_hbm.shape[0] // _NW
    base = wid * rows_per_w
    nchunk = rows_per_w // _CHUNK
    pltpu.sync_copy(const_hbm, constv)
    const_regs = [constv[pl.ds(j * 16, 16)] for j in range(8)]

    def in_slice(idx):
        return cube_hbm.at[pl.ds(base + idx * _CHUNK, _CHUNK), :]

    def out_slice(idx):
        return out_hbm.at[pl.ds(base + idx * _CHUNK, _CHUNK), :]

    for b in range(_PF):
        pltpu.async_copy(in_slice(b), bufs[b], in_sems[b])

    def super_body(i):
        for b in range(_NBUF):
            idx = i + b
            pf = idx + _PF
            bpf = (b + _PF) % _NBUF

            @pl.when(pf >= _NBUF)
            def _():
                pltpu.make_async_copy(
                    bufs[bpf], out_slice(0), out_sems[bpf]).wait()

            @pl.when(pf < nchunk)
            def _():
                pltpu.async_copy(in_slice(pf), bufs[bpf], in_sems[bpf])

            pltpu.make_async_copy(in_slice(0), bufs[b], in_sems[b]).wait()
            plsc.parallel_loop(0, _CHUNK, 1, unroll=4)(
                lambda r: _row_pass(bufs[b], const_regs, r))
            pltpu.async_copy(bufs[b], out_slice(idx), out_sems[b])

    pl.loop(0, nchunk, step=_NBUF)(super_body)
    for b in range(_NBUF - _PF, _NBUF):
        pltpu.make_async_copy(bufs[b], out_slice(0), out_sems[b]).wait()


def _sc_call(flat, const):
    mesh = plsc.VectorSubcoreMesh(
        core_axis_name="c", subcore_axis_name="s",
        num_cores=_NC, num_subcores=_NS,
    )
    return pl.kernel(
        _sc_body,
        out_type=jax.ShapeDtypeStruct((_SC_ROWS, _C), jnp.float32),
        mesh=mesh,
        compiler_params=pltpu.CompilerParams(needs_layout_passes=False),
        scratch_types=[
            [pltpu.VMEM((_CHUNK, _C), jnp.float32) for _ in range(_NBUF)],
            pltpu.VMEM((_C,), jnp.float32),
            [pltpu.SemaphoreType.DMA for _ in range(_NBUF)],
            [pltpu.SemaphoreType.DMA for _ in range(_NBUF)],
        ],
    )(flat, const)


def _tc_body(cube_ref, const_ref, out_ref):
    x = cube_ref[...]
    keep = jnp.max(x, axis=-1, keepdims=True) == 1.0
    out_ref[...] = jnp.where(keep, x, const_ref[...])


def _tc_call(flat, const2d):
    rows = flat.shape[0]
    skip = _SC_ROWS // _TC_BLOCK
    grid = (rows - _SC_ROWS) // _TC_BLOCK
    return pl.pallas_call(
        _tc_body,
        grid=(grid,),
        in_specs=[
            pl.BlockSpec((_TC_BLOCK, _C), lambda i, s=skip: (i + s, 0)),
            pl.BlockSpec((1, _C), lambda i: (0, 0)),
        ],
        out_specs=pl.BlockSpec((_TC_BLOCK, _C), lambda i, s=skip: (i + s, 0)),
        out_shape=jax.ShapeDtypeStruct((rows, _C), flat.dtype),
    )(flat, const2d)


def kernel(cayley_cube, constant_distribution):
    b, n, _, c = cayley_cube.shape
    rows = b * n * n
    flat = cayley_cube.reshape(rows, c)
    sc_out = _sc_call(flat, constant_distribution)
    tc_out = _tc_call(flat, constant_distribution.reshape(1, c))
    out = lax.dynamic_update_slice(tc_out, sc_out, (0, 0))
    return out.reshape(b, n, n, c)


# speedup vs baseline: 1.9027x; 1.0247x over previous
import jax
import jax.numpy as jnp
from jax import lax
from jax.experimental import pallas as pl
from jax.experimental.pallas import tpu as pltpu
from jax.experimental.pallas import tpu_sc as plsc

_C = 128
_NC, _NS = 2, 16
_NW = _NC * _NS
_CHUNK = 128
_NBUF = 4
_PF = 2
_SC_ROWS = 65536

_TC_BLOCK = 16384


def _row_pass(buf, const_regs, r):
    xs = [buf[r, pl.ds(j * 16, 16)] for j in range(8)]
    m = xs[0]
    for j in range(1, 8):
        m = jnp.maximum(m, xs[j])
    n_le = plsc.all_reduce_population_count(m <= 1.0)
    n_eq = plsc.all_reduce_population_count(m == 1.0)
    keep16 = jnp.logical_and(n_le == 16, n_eq > 0)
    for j in range(8):
        buf[r, pl.ds(j * 16, 16)] = jnp.where(keep16, xs[j], const_regs[j])


def _sc_body(cube_hbm, const_hbm, out_hbm, bufs, constv, in_sems, out_sems):
    wid = lax.axis_index("s") * _NC + lax.axis_index("c")
    rows_per_w = out_hbm.shape[0] // _NW
    base = wid * rows_per_w
    nchunk = rows_per_w // _CHUNK
    pltpu.sync_copy(const_hbm, constv)
    const_regs = [constv[pl.ds(j * 16, 16)] for j in range(8)]

    def in_slice(idx):
        return cube_hbm.at[pl.ds(base + idx * _CHUNK, _CHUNK), :]

    def out_slice(idx):
        return out_hbm.at[pl.ds(base + idx * _CHUNK, _CHUNK), :]

    for b in range(_PF):
        pltpu.async_copy(in_slice(b), bufs[b], in_sems[b])

    def super_body(i):
        for b in range(_NBUF):
            idx = i + b
            pf = idx + _PF
            bpf = (b + _PF) % _NBUF

            @pl.when(pf >= _NBUF)
            def _():
                pltpu.make_async_copy(
                    bufs[bpf], out_slice(0), out_sems[bpf]).wait()

            @pl.when(pf < nchunk)
            def _():
                pltpu.async_copy(in_slice(pf), bufs[bpf], in_sems[bpf])

            pltpu.make_async_copy(in_slice(0), bufs[b], in_sems[b]).wait()
            plsc.parallel_loop(0, _CHUNK, 1, unroll=4)(
                lambda r: _row_pass(bufs[b], const_regs, r))
            pltpu.async_copy(bufs[b], out_slice(idx), out_sems[b])

    pl.loop(0, nchunk, step=_NBUF)(super_body)
    for b in range(_NBUF - _PF, _NBUF):
        pltpu.make_async_copy(bufs[b], out_slice(0), out_sems[b]).wait()


def _sc_call(flat, const):
    mesh = plsc.VectorSubcoreMesh(
        core_axis_name="c", subcore_axis_name="s",
        num_cores=_NC, num_subcores=_NS,
    )
    return pl.kernel(
        _sc_body,
        out_type=jax.ShapeDtypeStruct((_SC_ROWS, _C), jnp.float32),
        mesh=mesh,
        compiler_params=pltpu.CompilerParams(needs_layout_passes=False),
        scratch_types=[
            [pltpu.VMEM((_CHUNK, _C), jnp.float32) for _ in range(_NBUF)],
            pltpu.VMEM((_C,), jnp.float32),
            [pltpu.SemaphoreType.DMA for _ in range(_NBUF)],
            [pltpu.SemaphoreType.DMA for _ in range(_NBUF)],
        ],
    )(flat, const)


def _tc_body(cube_ref, const_ref, out_ref):
    x = cube_ref[...]
    keep = jnp.max(x, axis=-1, keepdims=True) == 1.0
    out_ref[...] = jnp.where(keep, x, const_ref[...])


def _tc_call(flat, const2d):
    rows = flat.shape[0]
    skip = _SC_ROWS // _TC_BLOCK
    grid = (rows - _SC_ROWS) // _TC_BLOCK
    return pl.pallas_call(
        _tc_body,
        grid=(grid,),
        in_specs=[
            pl.BlockSpec((_TC_BLOCK, _C), lambda i, s=skip: (i + s, 0)),
            pl.BlockSpec((1, _C), lambda i: (0, 0)),
        ],
        out_specs=pl.BlockSpec((_TC_BLOCK, _C), lambda i, s=skip: (i + s, 0)),
        out_shape=jax.ShapeDtypeStruct((rows, _C), flat.dtype),
    )(flat, const2d)


def kernel(cayley_cube, constant_distribution):
    b, n, _, c = cayley_cube.shape
    rows = b * n * n
    flat = cayley_cube.reshape(rows, c)
    tc_out = _tc_call(flat, constant_distribution.reshape(1, c))
    sc_out = _sc_call(flat, constant_distribution)
    out = lax.dynamic_update_slice(tc_out, sc_out, (0, 0))
    return out.reshape(b, n, n, c)


# speedup vs baseline: 2.0231x; 1.0633x over previous
import jax
import jax.numpy as jnp
from jax import lax
from jax.experimental import pallas as pl
from jax.experimental.pallas import tpu as pltpu
from jax.experimental.pallas import tpu_sc as plsc

_C = 128
_NC, _NS = 2, 16
_NW = _NC * _NS
_CHUNK = 64
_NBUF = 8
_PF = 4
_SC_ROWS = 262144

_TC_BLOCK = 16384


def _row_pass(buf, const_regs, r):
    xs = [buf[r, pl.ds(j * 16, 16)] for j in range(8)]
    m = xs[0]
    for j in range(1, 8):
        m = jnp.maximum(m, xs[j])
    n_le = plsc.all_reduce_population_count(m <= 1.0)
    n_eq = plsc.all_reduce_population_count(m == 1.0)
    keep16 = jnp.logical_and(n_le == 16, n_eq > 0)
    for j in range(8):
        buf[r, pl.ds(j * 16, 16)] = jnp.where(keep16, xs[j], const_regs[j])


def _sc_body(cube_hbm, const_hbm, out_hbm, bufs, constv, in_sems, out_sems):
    wid = lax.axis_index("s") * _NC + lax.axis_index("c")
    rows_per_w = out_hbm.shape[0] // _NW
    base = wid * rows_per_w
    nchunk = rows_per_w // _CHUNK
    pltpu.sync_copy(const_hbm, constv)
    const_regs = [constv[pl.ds(j * 16, 16)] for j in range(8)]

    def in_slice(idx):
        return cube_hbm.at[pl.ds(base + idx * _CHUNK, _CHUNK), :]

    def out_slice(idx):
        return out_hbm.at[pl.ds(base + idx * _CHUNK, _CHUNK), :]

    for b in range(_PF):
        pltpu.async_copy(in_slice(b), bufs[b], in_sems[b])

    def super_body(i):
        for b in range(_NBUF):
            idx = i + b
            pf = idx + _PF
            bpf = (b + _PF) % _NBUF

            @pl.when(pf >= _NBUF)
            def _():
                pltpu.make_async_copy(
                    bufs[bpf], out_slice(0), out_sems[bpf]).wait()

            @pl.when(pf < nchunk)
            def _():
                pltpu.async_copy(in_slice(pf), bufs[bpf], in_sems[bpf])

            pltpu.make_async_copy(in_slice(0), bufs[b], in_sems[b]).wait()
            plsc.parallel_loop(0, _CHUNK, 1, unroll=4)(
                lambda r: _row_pass(bufs[b], const_regs, r))
            pltpu.async_copy(bufs[b], out_slice(idx), out_sems[b])

    pl.loop(0, nchunk, step=_NBUF)(super_body)
    for b in range(_NBUF - _PF, _NBUF):
        pltpu.make_async_copy(bufs[b], out_slice(0), out_sems[b]).wait()


def _sc_call(flat, const):
    mesh = plsc.VectorSubcoreMesh(
        core_axis_name="c", subcore_axis_name="s",
        num_cores=_NC, num_subcores=_NS,
    )
    return pl.kernel(
        _sc_body,
        out_type=jax.ShapeDtypeStruct((_SC_ROWS, _C), jnp.float32),
        mesh=mesh,
        compiler_params=pltpu.CompilerParams(needs_layout_passes=False),
        scratch_types=[
            [pltpu.VMEM((_CHUNK, _C), jnp.float32) for _ in range(_NBUF)],
            pltpu.VMEM((_C,), jnp.float32),
            [pltpu.SemaphoreType.DMA for _ in range(_NBUF)],
            [pltpu.SemaphoreType.DMA for _ in range(_NBUF)],
        ],
    )(flat, const)


def _tc_body(cube_ref, const_ref, out_ref):
    x = cube_ref[...]
    keep = jnp.max(x, axis=-1, keepdims=True) == 1.0
    out_ref[...] = jnp.where(keep, x, const_ref[...])


def _tc_call(flat, const2d):
    rows = flat.shape[0]
    skip = _SC_ROWS // _TC_BLOCK
    grid = (rows - _SC_ROWS) // _TC_BLOCK
    return pl.pallas_call(
        _tc_body,
        grid=(grid,),
        in_specs=[
            pl.BlockSpec((_TC_BLOCK, _C), lambda i, s=skip: (i + s, 0)),
            pl.BlockSpec((1, _C), lambda i: (0, 0)),
        ],
        out_specs=pl.BlockSpec((_TC_BLOCK, _C), lambda i, s=skip: (i + s, 0)),
        out_shape=jax.ShapeDtypeStruct((rows, _C), flat.dtype),
    )(flat, const2d)


def kernel(cayley_cube, constant_distribution):
    b, n, _, c = cayley_cube.shape
    rows = b * n * n
    flat = cayley_cube.reshape(rows, c)
    out = _sc_call(flat, constant_distribution)
    return out.reshape(b, n, n, c)
